# baseline (speedup 1.0000x reference)
import functools

import jax
import jax.numpy as jnp
from jax.experimental import pallas as pl
from jax.experimental.pallas import tpu as pltpu
from jax.experimental.shard_map import shard_map
from jax.sharding import Mesh, PartitionSpec as P

_LANE = 128
_BN = 4


def _reduce_into(x_ref, t_ref, acc_i, acc_d):
    shape = x_ref.shape
    rows = shape[0] * shape[1] * shape[2]
    W = shape[3]
    x = x_ref[...].reshape(rows, W).astype(jnp.float32)
    t = t_ref[...].reshape(rows, W).astype(jnp.float32)

    s = 0.5 * jnp.tanh(0.5 * x) + 0.5
    pi = (s * t).reshape(rows // 8, 8, W).sum(axis=0)
    pd = (s + t).reshape(rows // 8, 8, W).sum(axis=0)

    for k in range(W // _LANE):
        acc_i = acc_i + pi[:, k * _LANE:(k + 1) * _LANE]
        acc_d = acc_d + pd[:, k * _LANE:(k + 1) * _LANE]
    return acc_i, acc_d


def _dice_body(xa_ref, xb_ref, ta_ref, tb_ref, o_ref):
    i = pl.program_id(0)

    @pl.when(i == 0)
    def _init():
        o_ref[...] = jnp.zeros_like(o_ref)

    acc_i = jnp.zeros((8, _LANE), jnp.float32)
    acc_d = jnp.zeros((8, _LANE), jnp.float32)
    acc_i, acc_d = _reduce_into(xa_ref, ta_ref, acc_i, acc_d)
    acc_i, acc_d = _reduce_into(xb_ref, tb_ref, acc_i, acc_d)

    o_ref[0] += acc_i
    o_ref[1] += acc_d


def _partial_sums(x, t):
    n, C, H, W = x.shape
    bn = _BN if n % _BN == 0 else n
    ni = n // bn

    def imap_a(i):
        return (i, 1, 0, 0)

    def imap_b(i):
        return (i, 1, 0, 0)

    spec_a = pl.BlockSpec((bn, 1, H, W), imap_a)
    spec_b = pl.BlockSpec((bn, C - 2, H, W), imap_b)

    out = pl.pallas_call(
        _dice_body,
        out_shape=jax.ShapeDtypeStruct((2, 8, _LANE), jnp.float32),
        grid_spec=pltpu.PrefetchScalarGridSpec(
            num_scalar_prefetch=0,
            grid=(ni,),
            in_specs=[spec_a, spec_b, spec_a, spec_b],
            out_specs=pl.BlockSpec((2, 8, _LANE), lambda i: (0, 0, 0)),
        ),
        compiler_params=pltpu.CompilerParams(
            dimension_semantics=("arbitrary",)),
    )(x, x, t, t)

    return jnp.sum(out.reshape(2, 8 * _LANE), axis=1)


def _sharded_sums(x, t):
    sums = _partial_sums(x, t)
    return jax.lax.psum(sums, "d")


@jax.jit
def kernel(inputs, targets):
    devs = jax.devices()
    if len(devs) >= 2 and inputs.shape[0] % 2 == 0:
        mesh = Mesh(devs[:2], ("d",))
        sums = shard_map(
            _sharded_sums, mesh=mesh,
            in_specs=(P("d"), P("d")), out_specs=P(),
            check_rep=False,
        )(inputs, targets)
    else:
        sums = _partial_sums(inputs, targets)
    one = jnp.float32(1.0)
    return (2.0 * sums[0] + one) / (sums[1] + one)


# speedup vs baseline: 30.9746x; 30.9746x over previous
import jax
import jax.numpy as jnp
from jax.experimental import pallas as pl
from jax.experimental.pallas import tpu as pltpu

_LANE = 128
_BN = 4


def _reduce_into(x_ref, t_ref, acc_i, acc_d):
    shape = x_ref.shape
    rows = shape[0] * shape[1] * shape[2]
    W = shape[3]
    x = x_ref[...].reshape(rows, W).astype(jnp.float32)
    t = t_ref[...].reshape(rows, W).astype(jnp.float32)

    s = 0.5 * jnp.tanh(0.5 * x) + 0.5
    pi = (s * t).reshape(rows // 8, 8, W).sum(axis=0)
    pd = (s + t).reshape(rows // 8, 8, W).sum(axis=0)

    for k in range(W // _LANE):
        acc_i = acc_i + pi[:, k * _LANE:(k + 1) * _LANE]
        acc_d = acc_d + pd[:, k * _LANE:(k + 1) * _LANE]
    return acc_i, acc_d


def _dice_body(xa_ref, xb_ref, ta_ref, tb_ref, o_ref):
    i = pl.program_id(0)

    @pl.when(i == 0)
    def _init():
        o_ref[...] = jnp.zeros_like(o_ref)

    acc_i = jnp.zeros((8, _LANE), jnp.float32)
    acc_d = jnp.zeros((8, _LANE), jnp.float32)
    acc_i, acc_d = _reduce_into(xa_ref, ta_ref, acc_i, acc_d)
    acc_i, acc_d = _reduce_into(xb_ref, tb_ref, acc_i, acc_d)

    o_ref[0] += acc_i
    o_ref[1] += acc_d


@jax.jit
def kernel(inputs, targets):
    N, C, H, W = inputs.shape
    bn = _BN if N % _BN == 0 else N
    ni = N // bn

    def imap_a(i):
        return (i, 1, 0, 0)

    def imap_b(i):
        return (i, 1, 0, 0)

    spec_a = pl.BlockSpec((bn, 1, H, W), imap_a)
    spec_b = pl.BlockSpec((bn, C - 2, H, W), imap_b)

    out = pl.pallas_call(
        _dice_body,
        out_shape=jax.ShapeDtypeStruct((2, 8, _LANE), jnp.float32),
        grid_spec=pltpu.PrefetchScalarGridSpec(
            num_scalar_prefetch=0,
            grid=(ni,),
            in_specs=[spec_a, spec_b, spec_a, spec_b],
            out_specs=pl.BlockSpec((2, 8, _LANE), lambda i: (0, 0, 0)),
        ),
        compiler_params=pltpu.CompilerParams(
            dimension_semantics=("arbitrary",)),
    )(inputs, inputs, targets, targets)

    sums = jnp.sum(out.reshape(2, 8 * _LANE), axis=1)
    one = jnp.float32(1.0)
    return (2.0 * sums[0] + one) / (sums[1] + one)
